# Initial kernel scaffold; baseline (speedup 1.0000x reference)
import jax
import jax.numpy as jnp
from jax.experimental import pallas as pl


def kernel(positions, edge_index, numbers, rcov, r4r2, c6_table, cn_ref):
    raise NotImplementedError("write your pallas kernel here")


# speedup vs baseline: 27.0263x; 27.0263x over previous
import functools

import jax
import jax.numpy as jnp
from jax import lax
from jax.experimental import pallas as pl
from jax.experimental.pallas import tpu as pltpu
from jax.experimental.pallas import tpu_sc as plsc

N = 50000
E = 800000
Z = 95
M = 7

NPAD = 51200
EPAD = 819200
CH = 128
EPT = EPAD // 32
NCH = EPT // CH
APT = NPAD // 32

_CP = pltpu.CompilerParams(use_tc_tiling_on_sc=False,
                           needs_layout_passes=False)
_MESH = plsc.VectorSubcoreMesh(core_axis_name="c", subcore_axis_name="s",
                               num_cores=2, num_subcores=16)


def _wid():
    return lax.axis_index("s") * 2 + lax.axis_index("c")


def _rsqrt(x):
    i = lax.bitcast_convert_type(x, jnp.int32)
    i = jnp.int32(0x5F3759DF) - lax.shift_right_logical(i, 1)
    y = lax.bitcast_convert_type(i, jnp.float32)
    for _ in range(3):
        y = y * (1.5 - 0.5 * x * y * y)
    return y


def _iota():
    return lax.iota(jnp.int32, 16)


def _c(v):
    return jnp.full((16,), v, jnp.int32)


def _p1_body(src_h, dst_h, rec1_h, r2_h, pidx_h, cn2_h,
             sidx, didx, reci, recj, cfb, r2b, pxb, zb, sem, cn_sh):
    wid = _wid()
    sid = lax.axis_index("s")
    cid = lax.axis_index("c")
    lanes = _iota()

    zseg = NPAD // 16
    for t in range(zseg // 16):
        zb[pl.ds(t * 16, 16)] = jnp.zeros((16,), jnp.float32)
    pltpu.sync_copy(zb, cn_sh.at[pl.ds(sid * zseg, zseg)])
    plsc.subcore_barrier()

    def chunk(i, carry):
        base = pl.multiple_of(wid * EPT + i * CH, CH)
        pltpu.sync_copy(src_h.at[pl.ds(base, CH)], sidx)
        pltpu.sync_copy(dst_h.at[pl.ds(base, CH)], didx)
        pltpu.async_copy(rec1_h.at[sidx], reci, sem).wait()
        pltpu.async_copy(rec1_h.at[didx], recj, sem).wait()
        for g in range(CH // 16):
            rows = lanes + g * 16
            xi = plsc.load_gather(reci, [rows, _c(0)])
            yi = plsc.load_gather(reci, [rows, _c(1)])
            zi_ = plsc.load_gather(reci, [rows, _c(2)])
            ri = plsc.load_gather(reci, [rows, _c(3)])
            ni = plsc.load_gather(reci, [rows, _c(4)])
            xj = plsc.load_gather(recj, [rows, _c(0)])
            yj = plsc.load_gather(recj, [rows, _c(1)])
            zj_ = plsc.load_gather(recj, [rows, _c(2)])
            rj = plsc.load_gather(recj, [rows, _c(3)])
            nj = plsc.load_gather(recj, [rows, _c(4)])
            dx = xi - xj
            dy = yi - yj
            dz = zi_ - zj_
            r2 = dx * dx + dy * dy + dz * dz + 1e-12
            invd = _rsqrt(r2)
            rc = ri + rj
            cf = 1.0 / (1.0 + jnp.exp(16.0 - 16.0 * rc * invd))
            gid = base + g * 16 + lanes
            live = (r2 < 625.0) & (gid < E)
            cf = jnp.where(live, cf, 0.0)
            cfb[pl.ds(g * 16, 16)] = cf
            r2b[pl.ds(g * 16, 16)] = r2
            pxb[pl.ds(g * 16, 16)] = (ni * 95.0 + nj).astype(jnp.int32)
        pltpu.sync_copy(cfb, cn_sh.at[sidx], add=True)
        pltpu.sync_copy(cfb, cn_sh.at[didx], add=True)
        pltpu.sync_copy(r2b, r2_h.at[pl.ds(base, CH)])
        pltpu.sync_copy(pxb, pidx_h.at[pl.ds(base, CH)])
        return carry

    lax.fori_loop(0, NCH, chunk, 0)
    plsc.subcore_barrier()
    pltpu.sync_copy(cn_sh.at[pl.ds(sid * zseg, zseg)], zb)
    pltpu.sync_copy(zb, cn2_h.at[pl.ds(cid * NPAD + sid * zseg, zseg)])


def _p15_body(cn2_h, num_h, cref_h, r4_h, rec2_h,
              cna, cnb, nums, cref, r4, rb):
    wid = _wid()
    lanes = _iota()
    abase = pl.multiple_of(wid * APT, APT)
    pltpu.sync_copy(cn2_h.at[pl.ds(abase, APT)], cna)
    pltpu.sync_copy(cn2_h.at[pl.ds(NPAD + abase, APT)], cnb)
    pltpu.sync_copy(num_h.at[pl.ds(abase, APT)], nums)
    pltpu.sync_copy(cref_h, cref)
    pltpu.sync_copy(r4_h, r4)

    def step(t, carry):
        cn = cna[pl.ds(t * 16, 16)] + cnb[pl.ds(t * 16, 16)]
        z = nums[pl.ds(t * 16, 16)]
        q = plsc.load_gather(r4, [z])
        sumw = jnp.zeros((16,), jnp.float32)
        for k in range(M):
            ref = plsc.load_gather(cref, [z * 8 + k])
            d = cn - ref
            w = jnp.where(ref >= 0.0, jnp.exp(-4.0 * d * d), 0.0)
            plsc.store_scatter(rb, [lanes * 16 + k], w)
            sumw = sumw + w
        plsc.store_scatter(rb, [lanes * 16 + 7], sumw)
        plsc.store_scatter(rb, [lanes * 16 + 8], q)
        pltpu.sync_copy(rb, rec2_h.at[pl.ds((abase + t * 16) * 16, 256)])
        return carry

    lax.fori_loop(0, APT // 16, step, 0)


def _p2_body(src_h, dst_h, r2_h, pidx_h, rec2_h, c6_h, out_h,
             sidx, didx, r2c, pxc, reci, recj, c6c, ob, sem):
    wid = _wid()
    lanes = _iota()

    def chunk(i, acc):
        base = pl.multiple_of(wid * EPT + i * CH, CH)
        pltpu.sync_copy(src_h.at[pl.ds(base, CH)], sidx)
        pltpu.sync_copy(dst_h.at[pl.ds(base, CH)], didx)
        pltpu.sync_copy(r2_h.at[pl.ds(base, CH)], r2c)
        pltpu.sync_copy(pidx_h.at[pl.ds(base, CH)], pxc)
        pltpu.async_copy(rec2_h.at[sidx], reci, sem).wait()
        pltpu.async_copy(rec2_h.at[didx], recj, sem).wait()
        pltpu.async_copy(c6_h.at[pxc], c6c, sem).wait()
        for g in range(CH // 16):
            rows = lanes + g * 16
            r2 = r2c[pl.ds(g * 16, 16)]
            swi = plsc.load_gather(reci, [rows, _c(7)])
            swj = plsc.load_gather(recj, [rows, _c(7)])
            qi = plsc.load_gather(reci, [rows, _c(8)])
            qj = plsc.load_gather(recj, [rows, _c(8)])
            wj = [plsc.load_gather(recj, [rows, _c(l)]) for l in range(M)]
            num = jnp.zeros((16,), jnp.float32)
            for k in range(M):
                wik = plsc.load_gather(reci, [rows, _c(k)])
                inner = jnp.zeros((16,), jnp.float32)
                for l in range(M):
                    c6v = plsc.load_gather(c6c, [rows, _c(k * 8 + l)])
                    inner = inner + wj[l] * c6v
                num = num + wik * inner
            den = jnp.maximum(swi * swj, 1e-12)
            c6ij = num / den
            qq = 3.0 * qi * qj
            c8ij = c6ij * qq
            r0 = qq * _rsqrt(qq)
            f = 0.4 * r0 + 4.8
            f2 = f * f
            f6 = f2 * f2 * f2
            f8 = f6 * f2
            d6 = r2 * r2 * r2
            d8 = d6 * r2
            e = -1.0 * c6ij / (d6 + f6) - 1.9 * c8ij / (d8 + f8)
            gid = base + g * 16 + lanes
            e = jnp.where((r2 < 2500.0) & (gid < E), e, 0.0)
            acc = acc + e
        return acc

    acc = lax.fori_loop(0, NCH, chunk, jnp.zeros((16,), jnp.float32))
    ob[...] = acc
    pltpu.sync_copy(ob, out_h.at[pl.ds(wid * 16, 16)])


def kernel(positions, edge_index, numbers, rcov, r4r2, c6_table, cn_ref):
    f32 = jnp.float32
    ep = jnp.pad(edge_index, ((0, 0), (0, EPAD - E)))
    src, dst = ep[0], ep[1]
    nump = jnp.pad(numbers, (0, NPAD - N))
    posp = jnp.pad(positions, ((0, NPAD - N), (0, 0)))
    rcov_a = rcov[nump]
    rec1 = jnp.concatenate(
        [posp, rcov_a[:, None],
         nump.astype(f32)[:, None],
         jnp.zeros((NPAD, 3), f32)], axis=1)
    c6p = jnp.pad(c6_table.reshape(Z * Z, M, M),
                  ((0, 0), (0, 1), (0, 1))).reshape(Z * Z, 64)
    crefp = jnp.pad(cn_ref, ((0, 0), (0, 1)), constant_values=-1.0).reshape(-1)
    r4p = jnp.pad(r4r2, (0, 1))

    p1 = pl.kernel(
        _p1_body,
        out_type=(jax.ShapeDtypeStruct((EPAD,), f32),
                  jax.ShapeDtypeStruct((EPAD,), jnp.int32),
                  jax.ShapeDtypeStruct((2 * NPAD,), f32)),
        mesh=_MESH,
        compiler_params=_CP,
        scratch_types=[
            pltpu.VMEM((CH,), jnp.int32),
            pltpu.VMEM((CH,), jnp.int32),
            pltpu.VMEM((CH, 8), f32),
            pltpu.VMEM((CH, 8), f32),
            pltpu.VMEM((CH,), f32),
            pltpu.VMEM((CH,), f32),
            pltpu.VMEM((CH,), jnp.int32),
            pltpu.VMEM((NPAD // 16,), f32),
            pltpu.SemaphoreType.DMA,
            pltpu.VMEM_SHARED((NPAD,), f32),
        ],
    )
    r2b, pidxb, cn2 = p1(src, dst, rec1)

    p15 = pl.kernel(
        _p15_body,
        out_type=jax.ShapeDtypeStruct((NPAD * 16,), f32),
        mesh=_MESH,
        compiler_params=_CP,
        scratch_types=[
            pltpu.VMEM((APT,), f32),
            pltpu.VMEM((APT,), f32),
            pltpu.VMEM((APT,), jnp.int32),
            pltpu.VMEM((Z * 8,), f32),
            pltpu.VMEM((Z + 1,), f32),
            pltpu.VMEM((256,), f32),
        ],
    )
    rec2 = p15(cn2, nump, crefp, r4p).reshape(NPAD, 16)

    p2 = pl.kernel(
        _p2_body,
        out_type=jax.ShapeDtypeStruct((32 * 16,), f32),
        mesh=_MESH,
        compiler_params=_CP,
        scratch_types=[
            pltpu.VMEM((CH,), jnp.int32),
            pltpu.VMEM((CH,), jnp.int32),
            pltpu.VMEM((CH,), f32),
            pltpu.VMEM((CH,), jnp.int32),
            pltpu.VMEM((CH, 16), f32),
            pltpu.VMEM((CH, 16), f32),
            pltpu.VMEM((CH, 64), f32),
            pltpu.VMEM((16,), f32),
            pltpu.SemaphoreType.DMA,
        ],
    )
    partials = p2(src, dst, r2b, pidxb, rec2, c6p)
    return jnp.sum(partials)


# speedup vs baseline: 33.2039x; 1.2286x over previous
import jax
import jax.numpy as jnp
from jax import lax
from jax.experimental import pallas as pl
from jax.experimental.pallas import tpu as pltpu
from jax.experimental.pallas import tpu_sc as plsc

N = 50000
E = 800000
Z = 95
M = 7

NPAD = 51200
EPAD = 819200
CH = 128
EPT = EPAD // 32
NCH = EPT // CH
APT = NPAD // 32
NS = 4

_CP = pltpu.CompilerParams(use_tc_tiling_on_sc=False,
                           needs_layout_passes=False)
_MESH = plsc.VectorSubcoreMesh(core_axis_name="c", subcore_axis_name="s",
                               num_cores=2, num_subcores=16)


def _wid():
    return lax.axis_index("s") * 2 + lax.axis_index("c")


def _rsqrt(x):
    i = lax.bitcast_convert_type(x, jnp.int32)
    i = jnp.int32(0x5F3759DF) - lax.shift_right_logical(i, 1)
    y = lax.bitcast_convert_type(i, jnp.float32)
    for _ in range(3):
        y = y * (1.5 - 0.5 * x * y * y)
    return y


def _iota():
    return lax.iota(jnp.int32, 16)


def _c(v):
    return jnp.full((16,), v, jnp.int32)


def _clamp(c):
    return jnp.minimum(c, NCH - 1)


def _p1_body(src_h, dst_h, rec1_h, r2_h, pidx_h, cn2_h,
             sidx, didx, reci, recj, cfb, r2b, pxb, zb, sem, cn_sh):
    wid = _wid()
    sid = lax.axis_index("s")
    cid = lax.axis_index("c")
    lanes = _iota()

    zseg = NPAD // 16
    for t in range(zseg // 16):
        zb[pl.ds(t * 16, 16)] = jnp.zeros((16,), jnp.float32)
    pltpu.sync_copy(zb, cn_sh.at[pl.ds(sid * zseg, zseg)])
    plsc.subcore_barrier()

    def chunk(i, carry):
        base = pl.multiple_of(wid * EPT + i * CH, CH)
        pltpu.sync_copy(src_h.at[pl.ds(base, CH)], sidx)
        pltpu.sync_copy(dst_h.at[pl.ds(base, CH)], didx)
        pltpu.async_copy(rec1_h.at[sidx], reci, sem).wait()
        pltpu.async_copy(rec1_h.at[didx], recj, sem).wait()
        for g in range(CH // 16):
            rows = lanes + g * 16
            xi = plsc.load_gather(reci, [rows, _c(0)])
            yi = plsc.load_gather(reci, [rows, _c(1)])
            zi_ = plsc.load_gather(reci, [rows, _c(2)])
            ri = plsc.load_gather(reci, [rows, _c(3)])
            ni = plsc.load_gather(reci, [rows, _c(4)])
            xj = plsc.load_gather(recj, [rows, _c(0)])
            yj = plsc.load_gather(recj, [rows, _c(1)])
            zj_ = plsc.load_gather(recj, [rows, _c(2)])
            rj = plsc.load_gather(recj, [rows, _c(3)])
            nj = plsc.load_gather(recj, [rows, _c(4)])
            dx = xi - xj
            dy = yi - yj
            dz = zi_ - zj_
            r2 = dx * dx + dy * dy + dz * dz + 1e-12
            invd = _rsqrt(r2)
            rc = ri + rj
            cf = 1.0 / (1.0 + jnp.exp(16.0 - 16.0 * rc * invd))
            gid = base + g * 16 + lanes
            live = (r2 < 625.0) & (gid < E)
            cf = jnp.where(live, cf, 0.0)
            cfb[pl.ds(g * 16, 16)] = cf
            r2b[pl.ds(g * 16, 16)] = r2
            pxb[pl.ds(g * 16, 16)] = (ni * 95.0 + nj).astype(jnp.int32)
        pltpu.sync_copy(cfb, cn_sh.at[sidx], add=True)
        pltpu.sync_copy(cfb, cn_sh.at[didx], add=True)
        pltpu.sync_copy(r2b, r2_h.at[pl.ds(base, CH)])
        pltpu.sync_copy(pxb, pidx_h.at[pl.ds(base, CH)])
        return carry

    lax.fori_loop(0, NCH, chunk, 0)
    plsc.subcore_barrier()
    pltpu.sync_copy(cn_sh.at[pl.ds(sid * zseg, zseg)], zb)
    pltpu.sync_copy(zb, cn2_h.at[pl.ds(cid * NPAD + sid * zseg, zseg)])


def _p15_body(cn2_h, num_h, cref_h, r4_h, rec2_h,
              cna, cnb, nums, cref, r4, rb):
    wid = _wid()
    lanes = _iota()
    abase = pl.multiple_of(wid * APT, APT)
    pltpu.sync_copy(cn2_h.at[pl.ds(abase, APT)], cna)
    pltpu.sync_copy(cn2_h.at[pl.ds(NPAD + abase, APT)], cnb)
    pltpu.sync_copy(num_h.at[pl.ds(abase, APT)], nums)
    pltpu.sync_copy(cref_h, cref)
    pltpu.sync_copy(r4_h, r4)

    def step(t, carry):
        cn = cna[pl.ds(t * 16, 16)] + cnb[pl.ds(t * 16, 16)]
        z = nums[pl.ds(t * 16, 16)]
        q = plsc.load_gather(r4, [z])
        sumw = jnp.zeros((16,), jnp.float32)
        for k in range(M):
            ref = plsc.load_gather(cref, [z * 8 + k])
            d = cn - ref
            w = jnp.where(ref >= 0.0, jnp.exp(-4.0 * d * d), 0.0)
            plsc.store_scatter(rb, [lanes * 16 + k], w)
            sumw = sumw + w
        plsc.store_scatter(rb, [lanes * 16 + 7], sumw)
        plsc.store_scatter(rb, [lanes * 16 + 8], q)
        pltpu.sync_copy(rb, rec2_h.at[pl.ds((abase + t * 16) * 16, 256)])
        return carry

    lax.fori_loop(0, APT // 16, step, 0)


def _p2_body(src_h, dst_h, r2_h, pidx_h, rec2_h, c6_h, out_h, *refs):
    sidx = refs[0:4]
    didx = refs[4:8]
    r2c = refs[8:12]
    pxc = refs[12:16]
    reci = refs[16:20]
    recj = refs[20:24]
    c6c = refs[24:28]
    ob = refs[28]
    linsem = refs[29:33]
    gatsem = refs[33:37]

    wid = _wid()
    lanes = _iota()

    def ebase(c):
        return pl.multiple_of(wid * EPT + c * CH, CH)

    def lin_start(s, c):
        b = ebase(c)
        pltpu.async_copy(src_h.at[pl.ds(b, CH)], sidx[s], linsem[s])
        pltpu.async_copy(dst_h.at[pl.ds(b, CH)], didx[s], linsem[s])
        pltpu.async_copy(r2_h.at[pl.ds(b, CH)], r2c[s], linsem[s])
        pltpu.async_copy(pidx_h.at[pl.ds(b, CH)], pxc[s], linsem[s])

    def lin_wait(s, c):
        b = ebase(c)
        pltpu.make_async_copy(src_h.at[pl.ds(b, CH)], sidx[s], linsem[s]).wait()
        pltpu.make_async_copy(dst_h.at[pl.ds(b, CH)], didx[s], linsem[s]).wait()
        pltpu.make_async_copy(r2_h.at[pl.ds(b, CH)], r2c[s], linsem[s]).wait()
        pltpu.make_async_copy(pidx_h.at[pl.ds(b, CH)], pxc[s], linsem[s]).wait()

    def gat_start(s):
        pltpu.async_copy(rec2_h.at[sidx[s]], reci[s], gatsem[s])
        pltpu.async_copy(rec2_h.at[didx[s]], recj[s], gatsem[s])
        pltpu.async_copy(c6_h.at[pxc[s]], c6c[s], gatsem[s])

    def gat_wait(s):
        pltpu.make_async_copy(rec2_h.at[sidx[s]], reci[s], gatsem[s]).wait()
        pltpu.make_async_copy(rec2_h.at[didx[s]], recj[s], gatsem[s]).wait()
        pltpu.make_async_copy(c6_h.at[pxc[s]], c6c[s], gatsem[s]).wait()

    def compute(s, c, acc):
        b = ebase(c)
        for g in range(CH // 16):
            rows = lanes + g * 16
            r2 = r2c[s][pl.ds(g * 16, 16)]
            swi = plsc.load_gather(reci[s], [rows, _c(7)])
            swj = plsc.load_gather(recj[s], [rows, _c(7)])
            qi = plsc.load_gather(reci[s], [rows, _c(8)])
            qj = plsc.load_gather(recj[s], [rows, _c(8)])
            wj = [plsc.load_gather(recj[s], [rows, _c(l)]) for l in range(M)]
            num = jnp.zeros((16,), jnp.float32)
            for k in range(M):
                wik = plsc.load_gather(reci[s], [rows, _c(k)])
                inner = jnp.zeros((16,), jnp.float32)
                for l in range(M):
                    c6v = plsc.load_gather(c6c[s], [rows, _c(k * 8 + l)])
                    inner = inner + wj[l] * c6v
                num = num + wik * inner
            den = jnp.maximum(swi * swj, 1e-12)
            c6ij = num / den
            qq = 3.0 * qi * qj
            c8ij = c6ij * qq
            r0 = qq * _rsqrt(qq)
            f = 0.4 * r0 + 4.8
            f2 = f * f
            f6 = f2 * f2 * f2
            f8 = f6 * f2
            d6 = r2 * r2 * r2
            d8 = d6 * r2
            e = -1.0 * c6ij / (d6 + f6) - 1.9 * c8ij / (d8 + f8)
            gid = b + g * 16 + lanes
            e = jnp.where((r2 < 2500.0) & (gid < E), e, 0.0)
            acc = acc + e
        return acc

    lin_start(0, 0)
    lin_wait(0, 0)
    gat_start(0)
    lin_start(1, 1)
    lin_start(2, 2)
    lin_start(3, 3)

    def body(i, acc):
        for s in range(NS):
            c = i * NS + s
            sn = (s + 1) % NS
            lin_wait(sn, _clamp(c + 1))
            gat_start(sn)
            gat_wait(s)
            acc = compute(s, c, acc)
            lin_start(s, _clamp(c + 4))
        return acc

    acc = lax.fori_loop(0, NCH // NS, body, jnp.zeros((16,), jnp.float32))
    gat_wait(0)
    lin_wait(1, NCH - 1)
    lin_wait(2, NCH - 1)
    lin_wait(3, NCH - 1)
    ob[...] = acc
    pltpu.sync_copy(ob, out_h.at[pl.ds(wid * 16, 16)])


def kernel(positions, edge_index, numbers, rcov, r4r2, c6_table, cn_ref):
    f32 = jnp.float32
    i32 = jnp.int32
    ep = jnp.pad(edge_index, ((0, 0), (0, EPAD - E)))
    src, dst = ep[0], ep[1]
    nump = jnp.pad(numbers, (0, NPAD - N))
    posp = jnp.pad(positions, ((0, NPAD - N), (0, 0)))
    rcov_a = rcov[nump]
    rec1 = jnp.concatenate(
        [posp, rcov_a[:, None],
         nump.astype(f32)[:, None],
         jnp.zeros((NPAD, 3), f32)], axis=1)
    c6p = jnp.pad(c6_table.reshape(Z * Z, M, M),
                  ((0, 0), (0, 1), (0, 1))).reshape(Z * Z, 64)
    crefp = jnp.pad(cn_ref, ((0, 0), (0, 1)), constant_values=-1.0).reshape(-1)
    r4p = jnp.pad(r4r2, (0, 1))

    sem = pltpu.SemaphoreType.DMA
    p1 = pl.kernel(
        _p1_body,
        out_type=(jax.ShapeDtypeStruct((EPAD,), f32),
                  jax.ShapeDtypeStruct((EPAD,), i32),
                  jax.ShapeDtypeStruct((2 * NPAD,), f32)),
        mesh=_MESH,
        compiler_params=_CP,
        scratch_types=[
            pltpu.VMEM((CH,), jnp.int32),
            pltpu.VMEM((CH,), jnp.int32),
            pltpu.VMEM((CH, 8), f32),
            pltpu.VMEM((CH, 8), f32),
            pltpu.VMEM((CH,), f32),
            pltpu.VMEM((CH,), f32),
            pltpu.VMEM((CH,), jnp.int32),
            pltpu.VMEM((NPAD // 16,), f32),
            pltpu.SemaphoreType.DMA,
            pltpu.VMEM_SHARED((NPAD,), f32),
        ],
    )
    r2b, pidxb, cn2 = p1(src, dst, rec1)

    p15 = pl.kernel(
        _p15_body,
        out_type=jax.ShapeDtypeStruct((NPAD * 16,), f32),
        mesh=_MESH,
        compiler_params=_CP,
        scratch_types=[
            pltpu.VMEM((APT,), f32),
            pltpu.VMEM((APT,), f32),
            pltpu.VMEM((APT,), i32),
            pltpu.VMEM((Z * 8,), f32),
            pltpu.VMEM((Z + 1,), f32),
            pltpu.VMEM((256,), f32),
        ],
    )
    rec2 = p15(cn2, nump, crefp, r4p).reshape(NPAD, 16)

    p2 = pl.kernel(
        _p2_body,
        out_type=jax.ShapeDtypeStruct((32 * 16,), f32),
        mesh=_MESH,
        compiler_params=_CP,
        scratch_types=(
            [pltpu.VMEM((CH,), i32)] * 8
            + [pltpu.VMEM((CH,), f32)] * 4
            + [pltpu.VMEM((CH,), i32)] * 4
            + [pltpu.VMEM((CH, 16), f32)] * 8
            + [pltpu.VMEM((CH, 64), f32)] * 4
            + [pltpu.VMEM((16,), f32)]
            + [sem] * 8
        ),
    )
    partials = p2(src, dst, r2b, pidxb, rec2, c6p)
    return jnp.sum(partials)


# speedup vs baseline: 49.1313x; 1.4797x over previous
import jax
import jax.numpy as jnp
from jax import lax
from jax.experimental import pallas as pl
from jax.experimental.pallas import tpu as pltpu
from jax.experimental.pallas import tpu_sc as plsc

N = 50000
E = 800000
Z = 95
M = 7

NPAD = 51200
EPAD = 819200
CH = 128
EPT = EPAD // 32
NCH = EPT // CH
APT = NPAD // 32
NS = 4

_CP = pltpu.CompilerParams(use_tc_tiling_on_sc=False,
                           needs_layout_passes=False)
_MESH = plsc.VectorSubcoreMesh(core_axis_name="c", subcore_axis_name="s",
                               num_cores=2, num_subcores=16)


def _wid():
    return lax.axis_index("s") * 2 + lax.axis_index("c")


def _rsqrt(x):
    i = lax.bitcast_convert_type(x, jnp.int32)
    i = jnp.int32(0x5F3759DF) - lax.shift_right_logical(i, 1)
    y = lax.bitcast_convert_type(i, jnp.float32)
    for _ in range(3):
        y = y * (1.5 - 0.5 * x * y * y)
    return y


def _iota():
    return lax.iota(jnp.int32, 16)


def _c(v):
    return jnp.full((16,), v, jnp.int32)


def _clamp(c):
    return jnp.minimum(c, NCH - 1)


def _p1_body(src_h, dst_h, rec1_h, r2_h, pidx_h, cn2_h, *refs):
    sidx = refs[0:4]
    didx = refs[4:8]
    reci = refs[8:12]
    recj = refs[12:16]
    cfb = refs[16]
    r2b = refs[17]
    pxb = refs[18]
    zb = refs[19]
    linsem = refs[20:24]
    gatsem = refs[24:28]
    cn_sh = refs[28]

    wid = _wid()
    sid = lax.axis_index("s")
    cid = lax.axis_index("c")
    lanes = _iota()

    zseg = NPAD // 16
    for t in range(zseg // 16):
        zb[pl.ds(t * 16, 16)] = jnp.zeros((16,), jnp.float32)
    pltpu.sync_copy(zb, cn_sh.at[pl.ds(sid * zseg, zseg)])
    plsc.subcore_barrier()

    def ebase(c):
        return pl.multiple_of(wid * EPT + c * CH, CH)

    def lin_start(s, c):
        b = ebase(c)
        pltpu.async_copy(src_h.at[pl.ds(b, CH)], sidx[s], linsem[s])
        pltpu.async_copy(dst_h.at[pl.ds(b, CH)], didx[s], linsem[s])

    def lin_wait(s, c):
        b = ebase(c)
        pltpu.make_async_copy(src_h.at[pl.ds(b, CH)], sidx[s], linsem[s]).wait()
        pltpu.make_async_copy(dst_h.at[pl.ds(b, CH)], didx[s], linsem[s]).wait()

    def gat_start(s):
        pltpu.async_copy(rec1_h.at[sidx[s]], reci[s], gatsem[s])
        pltpu.async_copy(rec1_h.at[didx[s]], recj[s], gatsem[s])

    def gat_wait(s):
        pltpu.make_async_copy(rec1_h.at[sidx[s]], reci[s], gatsem[s]).wait()
        pltpu.make_async_copy(rec1_h.at[didx[s]], recj[s], gatsem[s]).wait()

    def compute(s, c):
        b = ebase(c)
        for g in range(CH // 16):
            rows = lanes + g * 16
            xi = plsc.load_gather(reci[s], [rows, _c(0)])
            yi = plsc.load_gather(reci[s], [rows, _c(1)])
            zi_ = plsc.load_gather(reci[s], [rows, _c(2)])
            ri = plsc.load_gather(reci[s], [rows, _c(3)])
            ni = plsc.load_gather(reci[s], [rows, _c(4)])
            xj = plsc.load_gather(recj[s], [rows, _c(0)])
            yj = plsc.load_gather(recj[s], [rows, _c(1)])
            zj_ = plsc.load_gather(recj[s], [rows, _c(2)])
            rj = plsc.load_gather(recj[s], [rows, _c(3)])
            nj = plsc.load_gather(recj[s], [rows, _c(4)])
            dx = xi - xj
            dy = yi - yj
            dz = zi_ - zj_
            r2 = dx * dx + dy * dy + dz * dz + 1e-12
            invd = _rsqrt(r2)
            rc = ri + rj
            cf = 1.0 / (1.0 + jnp.exp(16.0 - 16.0 * rc * invd))
            gid = b + g * 16 + lanes
            live = (r2 < 625.0) & (gid < E)
            cf = jnp.where(live, cf, 0.0)
            cfb[pl.ds(g * 16, 16)] = cf
            r2b[pl.ds(s * CH + g * 16, 16)] = r2
            pxb[pl.ds(s * CH + g * 16, 16)] = (ni * 95.0 + nj).astype(jnp.int32)
        pltpu.sync_copy(cfb, cn_sh.at[sidx[s]], add=True)
        pltpu.sync_copy(cfb, cn_sh.at[didx[s]], add=True)

    lin_start(0, 0)
    lin_wait(0, 0)
    gat_start(0)
    lin_start(1, 1)
    lin_start(2, 2)
    lin_start(3, 3)

    def body(i, carry):
        for s in range(NS):
            c = i * NS + s
            sn = (s + 1) % NS
            lin_wait(sn, _clamp(c + 1))
            gat_start(sn)
            gat_wait(s)
            compute(s, c)
            lin_start(s, _clamp(c + 4))
        b0 = pl.multiple_of(wid * EPT + i * (NS * CH), NS * CH)
        pltpu.sync_copy(r2b, r2_h.at[pl.ds(b0, NS * CH)])
        pltpu.sync_copy(pxb, pidx_h.at[pl.ds(b0, NS * CH)])
        return carry

    lax.fori_loop(0, NCH // NS, body, 0)
    gat_wait(0)
    lin_wait(1, NCH - 1)
    lin_wait(2, NCH - 1)
    lin_wait(3, NCH - 1)
    plsc.subcore_barrier()
    pltpu.sync_copy(cn_sh.at[pl.ds(sid * zseg, zseg)], zb)
    pltpu.sync_copy(zb, cn2_h.at[pl.ds(cid * NPAD + sid * zseg, zseg)])


def _p15_body(cn2_h, num_h, cref_h, r4_h, rec2_h,
              cna, cnb, nums, cref, r4, rb):
    wid = _wid()
    lanes = _iota()
    abase = pl.multiple_of(wid * APT, APT)
    pltpu.sync_copy(cn2_h.at[pl.ds(abase, APT)], cna)
    pltpu.sync_copy(cn2_h.at[pl.ds(NPAD + abase, APT)], cnb)
    pltpu.sync_copy(num_h.at[pl.ds(abase, APT)], nums)
    pltpu.sync_copy(cref_h, cref)
    pltpu.sync_copy(r4_h, r4)

    def step(t, carry):
        cn = cna[pl.ds(t * 16, 16)] + cnb[pl.ds(t * 16, 16)]
        z = nums[pl.ds(t * 16, 16)]
        q = plsc.load_gather(r4, [z])
        sumw = jnp.zeros((16,), jnp.float32)
        for k in range(M):
            ref = plsc.load_gather(cref, [z * 8 + k])
            d = cn - ref
            w = jnp.where(ref >= 0.0, jnp.exp(-4.0 * d * d), 0.0)
            plsc.store_scatter(rb, [lanes * 16 + k], w)
            sumw = sumw + w
        plsc.store_scatter(rb, [lanes * 16 + 7], sumw)
        plsc.store_scatter(rb, [lanes * 16 + 8], q)
        pltpu.sync_copy(rb, rec2_h.at[pl.ds((abase + t * 16) * 16, 256)])
        return carry

    lax.fori_loop(0, APT // 16, step, 0)


def _p2_body(src_h, dst_h, r2_h, pidx_h, rec2_h, c6_h, out_h, *refs):
    sidx = refs[0:4]
    didx = refs[4:8]
    r2c = refs[8:12]
    pxc = refs[12:16]
    reci = refs[16:20]
    recj = refs[20:24]
    c6c = refs[24:28]
    ob = refs[28]
    linsem = refs[29:33]
    gatsem = refs[33:37]

    wid = _wid()
    lanes = _iota()

    def ebase(c):
        return pl.multiple_of(wid * EPT + c * CH, CH)

    def lin_start(s, c):
        b = ebase(c)
        pltpu.async_copy(src_h.at[pl.ds(b, CH)], sidx[s], linsem[s])
        pltpu.async_copy(dst_h.at[pl.ds(b, CH)], didx[s], linsem[s])
        pltpu.async_copy(r2_h.at[pl.ds(b, CH)], r2c[s], linsem[s])
        pltpu.async_copy(pidx_h.at[pl.ds(b, CH)], pxc[s], linsem[s])

    def lin_wait(s, c):
        b = ebase(c)
        pltpu.make_async_copy(src_h.at[pl.ds(b, CH)], sidx[s], linsem[s]).wait()
        pltpu.make_async_copy(dst_h.at[pl.ds(b, CH)], didx[s], linsem[s]).wait()
        pltpu.make_async_copy(r2_h.at[pl.ds(b, CH)], r2c[s], linsem[s]).wait()
        pltpu.make_async_copy(pidx_h.at[pl.ds(b, CH)], pxc[s], linsem[s]).wait()

    def gat_start(s):
        pltpu.async_copy(rec2_h.at[sidx[s]], reci[s], gatsem[s])
        pltpu.async_copy(rec2_h.at[didx[s]], recj[s], gatsem[s])
        pltpu.async_copy(c6_h.at[pxc[s]], c6c[s], gatsem[s])

    def gat_wait(s):
        pltpu.make_async_copy(rec2_h.at[sidx[s]], reci[s], gatsem[s]).wait()
        pltpu.make_async_copy(rec2_h.at[didx[s]], recj[s], gatsem[s]).wait()
        pltpu.make_async_copy(c6_h.at[pxc[s]], c6c[s], gatsem[s]).wait()

    def compute(s, c, acc):
        b = ebase(c)
        for g in range(CH // 16):
            rows = lanes + g * 16
            r2 = r2c[s][pl.ds(g * 16, 16)]
            swi = plsc.load_gather(reci[s], [rows, _c(7)])
            swj = plsc.load_gather(recj[s], [rows, _c(7)])
            qi = plsc.load_gather(reci[s], [rows, _c(8)])
            qj = plsc.load_gather(recj[s], [rows, _c(8)])
            wj = [plsc.load_gather(recj[s], [rows, _c(l)]) for l in range(M)]
            num = jnp.zeros((16,), jnp.float32)
            for k in range(M):
                wik = plsc.load_gather(reci[s], [rows, _c(k)])
                inner = jnp.zeros((16,), jnp.float32)
                for l in range(M):
                    c6v = plsc.load_gather(c6c[s], [rows, _c(k * 8 + l)])
                    inner = inner + wj[l] * c6v
                num = num + wik * inner
            den = jnp.maximum(swi * swj, 1e-12)
            c6ij = num / den
            qq = 3.0 * qi * qj
            c8ij = c6ij * qq
            r0 = qq * _rsqrt(qq)
            f = 0.4 * r0 + 4.8
            f2 = f * f
            f6 = f2 * f2 * f2
            f8 = f6 * f2
            d6 = r2 * r2 * r2
            d8 = d6 * r2
            e = -1.0 * c6ij / (d6 + f6) - 1.9 * c8ij / (d8 + f8)
            gid = b + g * 16 + lanes
            e = jnp.where((r2 < 2500.0) & (gid < E), e, 0.0)
            acc = acc + e
        return acc

    lin_start(0, 0)
    lin_wait(0, 0)
    gat_start(0)
    lin_start(1, 1)
    lin_start(2, 2)
    lin_start(3, 3)

    def body(i, acc):
        for s in range(NS):
            c = i * NS + s
            sn = (s + 1) % NS
            lin_wait(sn, _clamp(c + 1))
            gat_start(sn)
            gat_wait(s)
            acc = compute(s, c, acc)
            lin_start(s, _clamp(c + 4))
        return acc

    acc = lax.fori_loop(0, NCH // NS, body, jnp.zeros((16,), jnp.float32))
    gat_wait(0)
    lin_wait(1, NCH - 1)
    lin_wait(2, NCH - 1)
    lin_wait(3, NCH - 1)
    ob[...] = acc
    pltpu.sync_copy(ob, out_h.at[pl.ds(wid * 16, 16)])


def kernel(positions, edge_index, numbers, rcov, r4r2, c6_table, cn_ref):
    f32 = jnp.float32
    i32 = jnp.int32
    ep = jnp.pad(edge_index, ((0, 0), (0, EPAD - E)))
    src, dst = ep[0], ep[1]
    nump = jnp.pad(numbers, (0, NPAD - N))
    posp = jnp.pad(positions, ((0, NPAD - N), (0, 0)))
    rcov_a = rcov[nump]
    rec1 = jnp.concatenate(
        [posp, rcov_a[:, None],
         nump.astype(f32)[:, None],
         jnp.zeros((NPAD, 3), f32)], axis=1)
    c6p = jnp.pad(c6_table.reshape(Z * Z, M, M),
                  ((0, 0), (0, 0), (0, 1))).reshape(Z * Z, 56)
    crefp = jnp.pad(cn_ref, ((0, 0), (0, 1)), constant_values=-1.0).reshape(-1)
    r4p = jnp.pad(r4r2, (0, 1))

    sem = pltpu.SemaphoreType.DMA
    p1 = pl.kernel(
        _p1_body,
        out_type=(jax.ShapeDtypeStruct((EPAD,), f32),
                  jax.ShapeDtypeStruct((EPAD,), i32),
                  jax.ShapeDtypeStruct((2 * NPAD,), f32)),
        mesh=_MESH,
        compiler_params=_CP,
        scratch_types=(
            [pltpu.VMEM((CH,), i32)] * 8
            + [pltpu.VMEM((CH, 8), f32)] * 8
            + [pltpu.VMEM((CH,), f32)]
            + [pltpu.VMEM((NS * CH,), f32)]
            + [pltpu.VMEM((NS * CH,), i32)]
            + [pltpu.VMEM((NPAD // 16,), f32)]
            + [pltpu.SemaphoreType.DMA] * 8
            + [pltpu.VMEM_SHARED((NPAD,), f32)]
        ),
    )
    r2b, pidxb, cn2 = p1(src, dst, rec1)

    p15 = pl.kernel(
        _p15_body,
        out_type=jax.ShapeDtypeStruct((NPAD * 16,), f32),
        mesh=_MESH,
        compiler_params=_CP,
        scratch_types=[
            pltpu.VMEM((APT,), f32),
            pltpu.VMEM((APT,), f32),
            pltpu.VMEM((APT,), i32),
            pltpu.VMEM((Z * 8,), f32),
            pltpu.VMEM((Z + 1,), f32),
            pltpu.VMEM((256,), f32),
        ],
    )
    rec2 = p15(cn2, nump, crefp, r4p).reshape(NPAD, 16)

    p2 = pl.kernel(
        _p2_body,
        out_type=jax.ShapeDtypeStruct((32 * 16,), f32),
        mesh=_MESH,
        compiler_params=_CP,
        scratch_types=(
            [pltpu.VMEM((CH,), i32)] * 8
            + [pltpu.VMEM((CH,), f32)] * 4
            + [pltpu.VMEM((CH,), i32)] * 4
            + [pltpu.VMEM((CH, 16), f32)] * 8
            + [pltpu.VMEM((CH, 56), f32)] * 4
            + [pltpu.VMEM((16,), f32)]
            + [sem] * 8
        ),
    )
    partials = p2(src, dst, r2b, pidxb, rec2, c6p)
    return jnp.sum(partials)


# speedup vs baseline: 51.2820x; 1.0438x over previous
import jax
import jax.numpy as jnp
from jax import lax
from jax.experimental import pallas as pl
from jax.experimental.pallas import tpu as pltpu
from jax.experimental.pallas import tpu_sc as plsc

N = 50000
E = 800000
Z = 95
M = 7

NPAD = 51200
EPAD = 819200
CH = 128
EPT = EPAD // 32
NCH = EPT // CH
APT = NPAD // 32
NS = 4

_CP = pltpu.CompilerParams(use_tc_tiling_on_sc=False,
                           needs_layout_passes=False)
_MESH = plsc.VectorSubcoreMesh(core_axis_name="c", subcore_axis_name="s",
                               num_cores=2, num_subcores=16)


def _wid():
    return lax.axis_index("s") * 2 + lax.axis_index("c")


def _rsqrt(x):
    i = lax.bitcast_convert_type(x, jnp.int32)
    i = jnp.int32(0x5F3759DF) - lax.shift_right_logical(i, 1)
    y = lax.bitcast_convert_type(i, jnp.float32)
    for _ in range(3):
        y = y * (1.5 - 0.5 * x * y * y)
    return y


def _iota():
    return lax.iota(jnp.int32, 16)


def _c(v):
    return jnp.full((16,), v, jnp.int32)


def _clamp(c):
    return jnp.minimum(c, NCH - 1)


def _p1_body(src_h, dst_h, rec1_h, r2_h, pidx_h, cn2_h, *refs):
    sidx = refs[0:4]
    didx = refs[4:8]
    reci = refs[8:12]
    recj = refs[12:16]
    cfb = refs[16]
    r2b = refs[17]
    pxb = refs[18]
    zb = refs[19]
    linsem = refs[20:24]
    gatsem = refs[24:28]
    cn_sh = refs[28]

    wid = _wid()
    sid = lax.axis_index("s")
    cid = lax.axis_index("c")
    lanes = _iota()

    zseg = NPAD // 16
    for t in range(zseg // 16):
        zb[pl.ds(t * 16, 16)] = jnp.zeros((16,), jnp.float32)
    pltpu.sync_copy(zb, cn_sh.at[pl.ds(sid * zseg, zseg)])
    plsc.subcore_barrier()

    def ebase(c):
        return pl.multiple_of(wid * EPT + c * CH, CH)

    def lin_start(s, c):
        b = ebase(c)
        pltpu.async_copy(src_h.at[pl.ds(b, CH)], sidx[s], linsem[s])
        pltpu.async_copy(dst_h.at[pl.ds(b, CH)], didx[s], linsem[s])

    def lin_wait(s, c):
        b = ebase(c)
        pltpu.make_async_copy(src_h.at[pl.ds(b, CH)], sidx[s], linsem[s]).wait()
        pltpu.make_async_copy(dst_h.at[pl.ds(b, CH)], didx[s], linsem[s]).wait()

    def gat_start(s):
        pltpu.async_copy(rec1_h.at[sidx[s]], reci[s], gatsem[s])
        pltpu.async_copy(rec1_h.at[didx[s]], recj[s], gatsem[s])

    def gat_wait(s):
        pltpu.make_async_copy(rec1_h.at[sidx[s]], reci[s], gatsem[s]).wait()
        pltpu.make_async_copy(rec1_h.at[didx[s]], recj[s], gatsem[s]).wait()

    def compute(s, c):
        b = ebase(c)
        for g in range(CH // 16):
            rows = lanes + g * 16
            xi = plsc.load_gather(reci[s], [rows, _c(0)])
            yi = plsc.load_gather(reci[s], [rows, _c(1)])
            zi_ = plsc.load_gather(reci[s], [rows, _c(2)])
            ri = plsc.load_gather(reci[s], [rows, _c(3)])
            ni = plsc.load_gather(reci[s], [rows, _c(4)])
            xj = plsc.load_gather(recj[s], [rows, _c(0)])
            yj = plsc.load_gather(recj[s], [rows, _c(1)])
            zj_ = plsc.load_gather(recj[s], [rows, _c(2)])
            rj = plsc.load_gather(recj[s], [rows, _c(3)])
            nj = plsc.load_gather(recj[s], [rows, _c(4)])
            dx = xi - xj
            dy = yi - yj
            dz = zi_ - zj_
            r2 = dx * dx + dy * dy + dz * dz + 1e-12
            invd = _rsqrt(r2)
            rc = ri + rj
            cf = 1.0 / (1.0 + jnp.exp(16.0 - 16.0 * rc * invd))
            gid = b + g * 16 + lanes
            live = (r2 < 625.0) & (gid < E)
            cf = jnp.where(live, cf, 0.0)
            cfb[pl.ds(g * 16, 16)] = cf
            r2b[pl.ds(s * CH + g * 16, 16)] = r2
            pxb[pl.ds(s * CH + g * 16, 16)] = (ni * 95.0 + nj).astype(jnp.int32)
        pltpu.sync_copy(cfb, cn_sh.at[sidx[s]], add=True)
        pltpu.sync_copy(cfb, cn_sh.at[didx[s]], add=True)

    lin_start(0, 0)
    lin_wait(0, 0)
    gat_start(0)
    lin_start(1, 1)
    lin_start(2, 2)
    lin_start(3, 3)

    def body(i, carry):
        for s in range(NS):
            c = i * NS + s
            sn = (s + 1) % NS
            lin_wait(sn, _clamp(c + 1))
            gat_start(sn)
            gat_wait(s)
            compute(s, c)
            lin_start(s, _clamp(c + 4))
        b0 = pl.multiple_of(wid * EPT + i * (NS * CH), NS * CH)
        pltpu.sync_copy(r2b, r2_h.at[pl.ds(b0, NS * CH)])
        pltpu.sync_copy(pxb, pidx_h.at[pl.ds(b0, NS * CH)])
        return carry

    lax.fori_loop(0, NCH // NS, body, 0)
    gat_wait(0)
    lin_wait(1, NCH - 1)
    lin_wait(2, NCH - 1)
    lin_wait(3, NCH - 1)
    plsc.subcore_barrier()
    pltpu.sync_copy(cn_sh.at[pl.ds(sid * zseg, zseg)], zb)
    pltpu.sync_copy(zb, cn2_h.at[pl.ds(cid * NPAD + sid * zseg, zseg)])


def _p15_body(cn2_h, num_h, cref_h, r4_h, rec2_h,
              cna, cnb, nums, cref, r4, rb):
    wid = _wid()
    lanes = _iota()
    abase = pl.multiple_of(wid * APT, APT)
    pltpu.sync_copy(cn2_h.at[pl.ds(abase, APT)], cna)
    pltpu.sync_copy(cn2_h.at[pl.ds(NPAD + abase, APT)], cnb)
    pltpu.sync_copy(num_h.at[pl.ds(abase, APT)], nums)
    pltpu.sync_copy(cref_h, cref)
    pltpu.sync_copy(r4_h, r4)

    def step(t, carry):
        cn = cna[pl.ds(t * 16, 16)] + cnb[pl.ds(t * 16, 16)]
        z = nums[pl.ds(t * 16, 16)]
        q = plsc.load_gather(r4, [z])
        sumw = jnp.zeros((16,), jnp.float32)
        ws = []
        for k in range(M):
            ref = plsc.load_gather(cref, [z * 8 + k])
            d = cn - ref
            w = jnp.where(ref >= 0.0, jnp.exp(-4.0 * d * d), 0.0)
            ws.append(w)
            sumw = sumw + w
        recip = 1.0 / jnp.maximum(sumw, 1e-12)
        for k in range(M):
            plsc.store_scatter(rb, [lanes * 8 + k], ws[k] * recip)
        plsc.store_scatter(rb, [lanes * 8 + 7], q)
        pltpu.sync_copy(rb, rec2_h.at[pl.ds((abase + t * 16) * 8, 128)])
        return carry

    lax.fori_loop(0, APT // 16, step, 0)


def _p2_body(src_h, dst_h, r2_h, pidx_h, rec2_h, c6_h, out_h, *refs):
    sidx = refs[0:4]
    didx = refs[4:8]
    r2c = refs[8:12]
    pxc = refs[12:16]
    reci = refs[16:20]
    recj = refs[20:24]
    c6c = refs[24:28]
    ob = refs[28]
    linsem = refs[29:33]
    gatsem = refs[33:37]

    wid = _wid()
    lanes = _iota()

    def ebase(c):
        return pl.multiple_of(wid * EPT + c * CH, CH)

    def lin_start(s, c):
        b = ebase(c)
        pltpu.async_copy(src_h.at[pl.ds(b, CH)], sidx[s], linsem[s])
        pltpu.async_copy(dst_h.at[pl.ds(b, CH)], didx[s], linsem[s])
        pltpu.async_copy(r2_h.at[pl.ds(b, CH)], r2c[s], linsem[s])
        pltpu.async_copy(pidx_h.at[pl.ds(b, CH)], pxc[s], linsem[s])

    def lin_wait(s, c):
        b = ebase(c)
        pltpu.make_async_copy(src_h.at[pl.ds(b, CH)], sidx[s], linsem[s]).wait()
        pltpu.make_async_copy(dst_h.at[pl.ds(b, CH)], didx[s], linsem[s]).wait()
        pltpu.make_async_copy(r2_h.at[pl.ds(b, CH)], r2c[s], linsem[s]).wait()
        pltpu.make_async_copy(pidx_h.at[pl.ds(b, CH)], pxc[s], linsem[s]).wait()

    def gat_start(s):
        pltpu.async_copy(rec2_h.at[sidx[s]], reci[s], gatsem[s])
        pltpu.async_copy(rec2_h.at[didx[s]], recj[s], gatsem[s])
        pltpu.async_copy(c6_h.at[pxc[s]], c6c[s], gatsem[s])

    def gat_wait(s):
        pltpu.make_async_copy(rec2_h.at[sidx[s]], reci[s], gatsem[s]).wait()
        pltpu.make_async_copy(rec2_h.at[didx[s]], recj[s], gatsem[s]).wait()
        pltpu.make_async_copy(c6_h.at[pxc[s]], c6c[s], gatsem[s]).wait()

    def compute(s, c, acc):
        b = ebase(c)
        for g in range(CH // 16):
            rows = lanes + g * 16
            r2 = r2c[s][pl.ds(g * 16, 16)]
            qi = plsc.load_gather(reci[s], [rows, _c(7)])
            qj = plsc.load_gather(recj[s], [rows, _c(7)])
            wj = [plsc.load_gather(recj[s], [rows, _c(l)]) for l in range(M)]
            num = jnp.zeros((16,), jnp.float32)
            for k in range(M):
                wik = plsc.load_gather(reci[s], [rows, _c(k)])
                inner = jnp.zeros((16,), jnp.float32)
                for l in range(M):
                    c6v = plsc.load_gather(c6c[s], [rows, _c(k * 8 + l)])
                    inner = inner + wj[l] * c6v
                num = num + wik * inner
            c6ij = num
            qq = 3.0 * qi * qj
            c8ij = c6ij * qq
            r0 = qq * _rsqrt(qq)
            f = 0.4 * r0 + 4.8
            f2 = f * f
            f6 = f2 * f2 * f2
            f8 = f6 * f2
            d6 = r2 * r2 * r2
            d8 = d6 * r2
            e = -1.0 * c6ij / (d6 + f6) - 1.9 * c8ij / (d8 + f8)
            gid = b + g * 16 + lanes
            e = jnp.where((r2 < 2500.0) & (gid < E), e, 0.0)
            acc = acc + e
        return acc

    lin_start(0, 0)
    lin_wait(0, 0)
    gat_start(0)
    lin_start(1, 1)
    lin_start(2, 2)
    lin_start(3, 3)

    def body(i, acc):
        for s in range(NS):
            c = i * NS + s
            sn = (s + 1) % NS
            lin_wait(sn, _clamp(c + 1))
            gat_start(sn)
            gat_wait(s)
            acc = compute(s, c, acc)
            lin_start(s, _clamp(c + 4))
        return acc

    acc = lax.fori_loop(0, NCH // NS, body, jnp.zeros((16,), jnp.float32))
    gat_wait(0)
    lin_wait(1, NCH - 1)
    lin_wait(2, NCH - 1)
    lin_wait(3, NCH - 1)
    ob[...] = acc
    pltpu.sync_copy(ob, out_h.at[pl.ds(wid * 16, 16)])


def kernel(positions, edge_index, numbers, rcov, r4r2, c6_table, cn_ref):
    f32 = jnp.float32
    i32 = jnp.int32
    ep = jnp.pad(edge_index, ((0, 0), (0, EPAD - E)))
    src, dst = ep[0], ep[1]
    nump = jnp.pad(numbers, (0, NPAD - N))
    posp = jnp.pad(positions, ((0, NPAD - N), (0, 0)))
    rcov_a = rcov[nump]
    rec1 = jnp.concatenate(
        [posp, rcov_a[:, None],
         nump.astype(f32)[:, None],
         jnp.zeros((NPAD, 3), f32)], axis=1)
    c6p = jnp.pad(c6_table.reshape(Z * Z, M, M),
                  ((0, 0), (0, 0), (0, 1))).reshape(Z * Z, 56)
    crefp = jnp.pad(cn_ref, ((0, 0), (0, 1)), constant_values=-1.0).reshape(-1)
    r4p = jnp.pad(r4r2, (0, 1))

    sem = pltpu.SemaphoreType.DMA
    p1 = pl.kernel(
        _p1_body,
        out_type=(jax.ShapeDtypeStruct((EPAD,), f32),
                  jax.ShapeDtypeStruct((EPAD,), i32),
                  jax.ShapeDtypeStruct((2 * NPAD,), f32)),
        mesh=_MESH,
        compiler_params=_CP,
        scratch_types=(
            [pltpu.VMEM((CH,), i32)] * 8
            + [pltpu.VMEM((CH, 8), f32)] * 8
            + [pltpu.VMEM((CH,), f32)]
            + [pltpu.VMEM((NS * CH,), f32)]
            + [pltpu.VMEM((NS * CH,), i32)]
            + [pltpu.VMEM((NPAD // 16,), f32)]
            + [pltpu.SemaphoreType.DMA] * 8
            + [pltpu.VMEM_SHARED((NPAD,), f32)]
        ),
    )
    r2b, pidxb, cn2 = p1(src, dst, rec1)

    p15 = pl.kernel(
        _p15_body,
        out_type=jax.ShapeDtypeStruct((NPAD * 8,), f32),
        mesh=_MESH,
        compiler_params=_CP,
        scratch_types=[
            pltpu.VMEM((APT,), f32),
            pltpu.VMEM((APT,), f32),
            pltpu.VMEM((APT,), i32),
            pltpu.VMEM((Z * 8,), f32),
            pltpu.VMEM((Z + 1,), f32),
            pltpu.VMEM((128,), f32),
        ],
    )
    rec2 = p15(cn2, nump, crefp, r4p).reshape(NPAD, 8)

    p2 = pl.kernel(
        _p2_body,
        out_type=jax.ShapeDtypeStruct((32 * 16,), f32),
        mesh=_MESH,
        compiler_params=_CP,
        scratch_types=(
            [pltpu.VMEM((CH,), i32)] * 8
            + [pltpu.VMEM((CH,), f32)] * 4
            + [pltpu.VMEM((CH,), i32)] * 4
            + [pltpu.VMEM((CH, 8), f32)] * 8
            + [pltpu.VMEM((CH, 56), f32)] * 4
            + [pltpu.VMEM((16,), f32)]
            + [sem] * 8
        ),
    )
    partials = p2(src, dst, r2b, pidxb, rec2, c6p)
    return jnp.sum(partials)
